# Initial kernel scaffold; baseline (speedup 1.0000x reference)
import jax
import jax.numpy as jnp
from jax.experimental import pallas as pl


def kernel(X, word_ebd, pos_ebd):
    raise NotImplementedError("write your pallas kernel here")


# speedup vs baseline: 1.4762x; 1.4762x over previous
import functools

import jax
import jax.numpy as jnp
from jax import lax
from jax.experimental import pallas as pl
from jax.experimental.pallas import tpu as pltpu
from jax.experimental.pallas import tpu_sc as plsc

B = 16384
L = 200
H = 256
V = 1000
N_TOKENS = B * L

_NC = 2
_NS = 16
_NW = _NC * _NS

CHUNK = 128
TOK_PER_W = N_TOKENS // _NW
CHUNKS_PER_W = TOK_PER_W // CHUNK
_LANES = 16
_VREGS_PER_ROW = H // _LANES


def _ebd_kernel(x_hbm, word_hbm, pos_hbm, out_hbm, idx_v, rows_v, pos_v, sem):
    wid = lax.axis_index("s") * _NC + lax.axis_index("c")
    w_base = wid * TOK_PER_W

    pltpu.sync_copy(pos_hbm.at[pl.ds(0, L)], pos_v)

    def chunk_body(c, carry):
        base = w_base + c * CHUNK
        pltpu.sync_copy(x_hbm.at[pl.ds(base, CHUNK)], idx_v)
        pltpu.async_copy(word_hbm.at[idx_v], rows_v, sem).wait()

        def row_body(i, carry2):
            l = lax.rem(base + i, L)
            for j in range(_VREGS_PER_ROW):
                sl = pl.ds(j * _LANES, _LANES)
                rows_v[i, sl] = rows_v[i, sl] + pos_v[l, sl]
            return carry2

        lax.fori_loop(0, CHUNK, row_body, 0, unroll=False)
        pltpu.sync_copy(rows_v, out_hbm.at[pl.ds(base, CHUNK)])
        return carry

    lax.fori_loop(0, CHUNKS_PER_W, chunk_body, 0, unroll=False)


@jax.jit
def _run(x_flat, word_ebd, pos_ebd):
    mesh = plsc.VectorSubcoreMesh(core_axis_name="c", subcore_axis_name="s")
    f = functools.partial(
        pl.kernel,
        mesh=mesh,
        out_type=jax.ShapeDtypeStruct((N_TOKENS, H), jnp.float32),
        scratch_types=[
            pltpu.VMEM((CHUNK,), jnp.int32),
            pltpu.VMEM((CHUNK, H), jnp.float32),
            pltpu.VMEM((L, H), jnp.float32),
            pltpu.SemaphoreType.DMA,
        ],
    )(_ebd_kernel)
    return f(x_flat, word_ebd, pos_ebd)


def kernel(X, word_ebd, pos_ebd):
    x_flat = X.reshape(-1).astype(jnp.int32)
    out = _run(x_flat, word_ebd, pos_ebd)
    return out.reshape(B, L, H)


# speedup vs baseline: 1.7435x; 1.1811x over previous
import functools

import jax
import jax.numpy as jnp
from jax import lax
from jax.experimental import pallas as pl
from jax.experimental.pallas import tpu as pltpu
from jax.experimental.pallas import tpu_sc as plsc

B = 16384
L = 200
H = 256
V = 1000
N_TOKENS = B * L

_NC = 2
_NS = 16
_NW = _NC * _NS

CHUNK = 128
TOK_PER_W = N_TOKENS // _NW
CHUNKS_PER_W = TOK_PER_W // CHUNK
_LANES = 16
_VREGS_PER_ROW = H // _LANES


def _add_pos_rows(rows_ref, pos_ref, lo, hi, pos_off):
    def body(i, carry):
        lrow = i + pos_off
        for j in range(_VREGS_PER_ROW):
            sl = pl.ds(j * _LANES, _LANES)
            rows_ref[i, sl] = rows_ref[i, sl] + pos_ref[lrow, sl]
        return carry

    lax.fori_loop(lo, hi, body, 0, unroll=False)


def _ebd_kernel(x_hbm, word_hbm, pos_hbm, out_hbm,
                idx0, idx1, rows0, rows1, pos_v,
                gsem0, gsem1, ssem0, ssem1):
    wid = lax.axis_index("s") * _NC + lax.axis_index("c")
    w_base = wid * TOK_PER_W
    idx = (idx0, idx1)
    rows = (rows0, rows1)
    gsem = (gsem0, gsem1)
    ssem = (ssem0, ssem1)

    pltpu.sync_copy(pos_hbm.at[pl.ds(0, L)], pos_v)

    pltpu.sync_copy(x_hbm.at[pl.ds(w_base, CHUNK)], idx0)
    pltpu.async_copy(word_hbm.at[idx0], rows0, gsem0)

    def outer(t, carry):
        c0 = t * 2
        for b in (0, 1):
            c = c0 + b
            nb = 1 - b
            base = w_base + c * CHUNK

            @pl.when(c < CHUNKS_PER_W - 1)
            def _start_next(b=b, nb=nb, c=c, base=base):
                @pl.when(c >= 1)
                def _drain_prev():
                    pltpu.make_async_copy(
                        rows[nb], out_hbm.at[pl.ds(0, CHUNK)], ssem[nb]
                    ).wait()

                pltpu.sync_copy(x_hbm.at[pl.ds(base + CHUNK, CHUNK)], idx[nb])
                pltpu.async_copy(word_hbm.at[idx[nb]], rows[nb], gsem[nb])

            pltpu.make_async_copy(word_hbm.at[idx[b]], rows[b], gsem[b]).wait()

            p = lax.rem(base, L)
            n1 = jnp.minimum(CHUNK, L - p)
            _add_pos_rows(rows[b], pos_v, 0, n1, p)
            _add_pos_rows(rows[b], pos_v, n1, CHUNK, p - L)

            pltpu.async_copy(rows[b], out_hbm.at[pl.ds(base, CHUNK)], ssem[b])
        return carry

    lax.fori_loop(0, CHUNKS_PER_W // 2, outer, 0, unroll=False)

    pltpu.make_async_copy(rows0, out_hbm.at[pl.ds(0, CHUNK)], ssem0).wait()
    pltpu.make_async_copy(rows1, out_hbm.at[pl.ds(0, CHUNK)], ssem1).wait()


@jax.jit
def _run(x_flat, word_ebd, pos_ebd):
    mesh = plsc.VectorSubcoreMesh(core_axis_name="c", subcore_axis_name="s")
    f = functools.partial(
        pl.kernel,
        mesh=mesh,
        out_type=jax.ShapeDtypeStruct((N_TOKENS, H), jnp.float32),
        scratch_types=[
            pltpu.VMEM((CHUNK,), jnp.int32),
            pltpu.VMEM((CHUNK,), jnp.int32),
            pltpu.VMEM((CHUNK, H), jnp.float32),
            pltpu.VMEM((CHUNK, H), jnp.float32),
            pltpu.VMEM((L, H), jnp.float32),
            pltpu.SemaphoreType.DMA,
            pltpu.SemaphoreType.DMA,
            pltpu.SemaphoreType.DMA,
            pltpu.SemaphoreType.DMA,
        ],
    )(_ebd_kernel)
    return f(x_flat, word_ebd, pos_ebd)


def kernel(X, word_ebd, pos_ebd):
    x_flat = X.reshape(-1).astype(jnp.int32)
    out = _run(x_flat, word_ebd, pos_ebd)
    return out.reshape(B, L, H)


# speedup vs baseline: 6.5195x; 3.7393x over previous
import functools

import jax
import jax.numpy as jnp
from jax import lax
from jax.experimental import pallas as pl
from jax.experimental.pallas import tpu as pltpu
from jax.experimental.pallas import tpu_sc as plsc

B = 16384
L = 200
H = 256
V = 1000
N_TOKENS = B * L

_NC = 2
_NS = 16
_NW = _NC * _NS

CHUNK = 128
TOK_PER_W = N_TOKENS // _NW
CHUNKS_PER_W = TOK_PER_W // CHUNK
_LANES = 16
_IDX_VREGS = CHUNK // _LANES


def _fuse_body(word_ref, pos_ref, out_ref):
    out_ref[...] = word_ref[...][None, :, :] + pos_ref[...][:, None, :]


def _build_fused(word_ebd, pos_ebd):
    lb = 8
    fused = pl.pallas_call(
        _fuse_body,
        grid=(L // lb,),
        in_specs=[
            pl.BlockSpec((V, H), lambda l: (0, 0)),
            pl.BlockSpec((lb, H), lambda l: (l, 0)),
        ],
        out_specs=pl.BlockSpec((lb, V, H), lambda l: (l, 0, 0)),
        out_shape=jax.ShapeDtypeStruct((L, V, H), jnp.float32),
    )(word_ebd, pos_ebd)
    return fused.reshape(L * V, H)


def _to_fused_rows(idx_ref, base):
    lane = lax.iota(jnp.int32, _LANES)
    p = lax.rem(base, L)
    for k in range(_IDX_VREGS):
        sl = pl.ds(k * _LANES, _LANES)
        t = lane + (p + k * _LANES)
        lmod = t - jnp.where(t >= L, L, 0)
        idx_ref[sl] = idx_ref[sl] + lmod * V


def _ebd_kernel(x_hbm, fused_hbm, out_hbm,
                idx0, idx1, rows0, rows1,
                gsem0, gsem1, ssem0, ssem1):
    wid = lax.axis_index("s") * _NC + lax.axis_index("c")
    w_base = wid * TOK_PER_W
    idx = (idx0, idx1)
    rows = (rows0, rows1)
    gsem = (gsem0, gsem1)
    ssem = (ssem0, ssem1)

    pltpu.sync_copy(x_hbm.at[pl.ds(w_base, CHUNK)], idx0)
    _to_fused_rows(idx0, w_base)
    pltpu.async_copy(fused_hbm.at[idx0], rows0, gsem0)

    def outer(t, carry):
        c0 = t * 2
        for b in (0, 1):
            c = c0 + b
            nb = 1 - b
            base = w_base + c * CHUNK

            @pl.when(c < CHUNKS_PER_W - 1)
            def _start_next(b=b, nb=nb, c=c, base=base):
                @pl.when(c >= 1)
                def _drain_prev():
                    pltpu.make_async_copy(
                        rows[nb], out_hbm.at[pl.ds(0, CHUNK)], ssem[nb]
                    ).wait()

                pltpu.sync_copy(x_hbm.at[pl.ds(base + CHUNK, CHUNK)], idx[nb])
                _to_fused_rows(idx[nb], base + CHUNK)
                pltpu.async_copy(fused_hbm.at[idx[nb]], rows[nb], gsem[nb])

            pltpu.make_async_copy(fused_hbm.at[idx[b]], rows[b], gsem[b]).wait()
            pltpu.async_copy(rows[b], out_hbm.at[pl.ds(base, CHUNK)], ssem[b])
        return carry

    lax.fori_loop(0, CHUNKS_PER_W // 2, outer, 0, unroll=False)

    pltpu.make_async_copy(rows0, out_hbm.at[pl.ds(0, CHUNK)], ssem0).wait()
    pltpu.make_async_copy(rows1, out_hbm.at[pl.ds(0, CHUNK)], ssem1).wait()


@jax.jit
def _run(x_flat, word_ebd, pos_ebd):
    fused = _build_fused(word_ebd, pos_ebd)
    mesh = plsc.VectorSubcoreMesh(core_axis_name="c", subcore_axis_name="s")
    f = functools.partial(
        pl.kernel,
        mesh=mesh,
        out_type=jax.ShapeDtypeStruct((N_TOKENS, H), jnp.float32),
        scratch_types=[
            pltpu.VMEM((CHUNK,), jnp.int32),
            pltpu.VMEM((CHUNK,), jnp.int32),
            pltpu.VMEM((CHUNK, H), jnp.float32),
            pltpu.VMEM((CHUNK, H), jnp.float32),
            pltpu.SemaphoreType.DMA,
            pltpu.SemaphoreType.DMA,
            pltpu.SemaphoreType.DMA,
            pltpu.SemaphoreType.DMA,
        ],
    )(_ebd_kernel)
    return f(x_flat, fused)


def kernel(X, word_ebd, pos_ebd):
    x_flat = X.reshape(-1).astype(jnp.int32)
    out = _run(x_flat, word_ebd, pos_ebd)
    return out.reshape(B, L, H)
